# Initial kernel scaffold; baseline (speedup 1.0000x reference)
import jax
import jax.numpy as jnp
from jax.experimental import pallas as pl


def kernel(loc_data, conf_data, priors, targets):
    raise NotImplementedError("write your pallas kernel here")


# speedup vs baseline: 16.4922x; 16.4922x over previous
import functools

import jax
import jax.numpy as jnp
from jax.experimental import pallas as pl
from jax.experimental.pallas import tpu as pltpu

_NUM_CLASSES = 21
_THRESHOLD = 0.5
_NEG_RATIO = 3
_VAR0 = 0.1
_VAR1 = 0.2
_MAX_FINITE_BITS = 0x7F7FFFFF


def _mbox_kernel(conf_ref, loc_ref, priors_ref, targets_ref,
                 out_l_ref, out_c_ref, lh_ref, npos_ref):
    b = pl.program_id(0)
    num = pl.num_programs(0)
    P = priors_ref.shape[1]

    @pl.when(b == 0)
    def _init():
        out_l_ref[...] = jnp.zeros((1, 1), jnp.float32)
        out_c_ref[...] = jnp.zeros((1, 1), jnp.float32)

    pr = priors_ref[...]
    cx = pr[0:1, :]
    cy = pr[1:2, :]
    w = pr[2:3, :]
    h = pr[3:4, :]
    px1 = cx - w / 2.0
    py1 = cy - h / 2.0
    px2 = cx + w / 2.0
    py2 = cy + h / 2.0
    area_b = (px2 - px1) * (py2 - py1)

    tg = targets_ref[0]
    tx1 = tg[:, 0:1]
    ty1 = tg[:, 1:2]
    tx2 = tg[:, 2:3]
    ty2 = tg[:, 3:4]
    tlab = tg[:, 4:5]
    area_a = (tx2 - tx1) * (ty2 - ty1)

    ix = jnp.clip(jnp.minimum(tx2, px2) - jnp.maximum(tx1, px1), 0.0, None)
    iy = jnp.clip(jnp.minimum(ty2, py2) - jnp.maximum(ty1, py1), 0.0, None)
    inter = ix * iy
    union = area_a + area_b - inter
    ov = inter / union

    T = ov.shape[0]
    jidx = jax.lax.broadcasted_iota(jnp.int32, ov.shape, 0)
    pidx = jax.lax.broadcasted_iota(jnp.int32, ov.shape, 1)

    bto = jnp.max(ov, axis=0, keepdims=True)
    bti = jnp.min(jnp.where(ov == bto, jidx, T), axis=0, keepdims=True)

    pmax = jnp.max(ov, axis=1, keepdims=True)
    bpi = jnp.min(jnp.where(ov == pmax, pidx, P), axis=1, keepdims=True)

    fmask = pidx == bpi
    forced = jnp.max(fmask.astype(jnp.int32), axis=0, keepdims=True) > 0
    forced_j = jnp.max(jnp.where(fmask, jidx, -1), axis=0, keepdims=True)

    bto = jnp.where(forced, 2.0, bto)
    bti = jnp.where(forced, forced_j, bti)

    onehot = jidx == bti

    def sel(col):
        return jnp.sum(jnp.where(onehot, col, 0.0), axis=0, keepdims=True)

    mx1 = sel(tx1)
    my1 = sel(ty1)
    mx2 = sel(tx2)
    my2 = sel(ty2)
    lab = sel(tlab)

    conf_lab = jnp.where(bto < _THRESHOLD, 0.0, lab)
    pos = conf_lab > 0.0
    conf_i = conf_lab.astype(jnp.int32)

    g_cx = ((mx1 + mx2) / 2.0 - cx) / (_VAR0 * w)
    g_cy = ((my1 + my2) / 2.0 - cy) / (_VAR0 * h)
    g_w = jnp.log((mx2 - mx1) / w) / _VAR1
    g_h = jnp.log((my2 - my1) / h) / _VAR1

    ld = loc_ref[0]

    def sl1(d):
        ad = jnp.abs(d)
        return jnp.where(ad < 1.0, 0.5 * d * d, ad - 0.5)

    s = (sl1(ld[0:1, :] - g_cx) + sl1(ld[1:2, :] - g_cy)
         + sl1(ld[2:3, :] - g_w) + sl1(ld[3:4, :] - g_h))
    out_l_ref[...] += jnp.sum(jnp.where(pos, s, 0.0), keepdims=True)

    cf = conf_ref[0]
    m = jnp.max(cf, axis=0, keepdims=True)
    sexp = jnp.sum(jnp.exp(cf - m), axis=0, keepdims=True)
    lse = m + jnp.log(sexp)
    cidx = jax.lax.broadcasted_iota(jnp.int32, cf.shape, 0)
    chosen = jnp.sum(jnp.where(cidx == conf_i, cf, 0.0), axis=0,
                     keepdims=True)
    ce = lse - chosen

    out_c_ref[...] += jnp.sum(jnp.where(pos, ce, 0.0), keepdims=True)
    npos_ref[pl.ds(b, 1), :] = jnp.sum(pos.astype(jnp.int32), axis=1,
                                       keepdims=True)
    lh_ref[pl.ds(b, 1), :] = jnp.where(pos, 0.0, ce)

    @pl.when(b == num - 1)
    def _finalize():
        lh = lh_ref[...]
        bits = jax.lax.bitcast_convert_type(lh, jnp.int32)
        npos = npos_ref[...]
        k = jnp.minimum(_NEG_RATIO * npos, P - 1)

        def body(_, carry):
            lo, hi = carry
            mid = lo + (hi - lo + 1) // 2
            cnt = jnp.sum((bits >= mid).astype(jnp.int32), axis=1,
                          keepdims=True)
            ok = cnt >= k
            return jnp.where(ok, mid, lo), jnp.where(ok, hi, mid - 1)

        lo0 = jnp.zeros_like(k)
        hi0 = jnp.full_like(k, _MAX_FINITE_BITS)
        lo, _ = jax.lax.fori_loop(0, 31, body, (lo0, hi0))
        gt = bits > lo
        cnt_gt = jnp.sum(gt.astype(jnp.int32), axis=1, keepdims=True)
        sum_gt = jnp.sum(jnp.where(gt, lh, 0.0), axis=1, keepdims=True)
        tval = jax.lax.bitcast_convert_type(lo, jnp.float32)
        topk = sum_gt + (k - cnt_gt).astype(jnp.float32) * tval

        n_total = jnp.sum(npos, keepdims=True).astype(jnp.float32)
        out_l_ref[...] = out_l_ref[...] / n_total
        out_c_ref[...] = (out_c_ref[...]
                          + jnp.sum(topk, axis=0, keepdims=True)) / n_total


@jax.jit
def kernel(loc_data, conf_data, priors, targets):
    B, P, C = conf_data.shape
    conf_t = jnp.transpose(conf_data, (0, 2, 1))
    loc_t = jnp.transpose(loc_data, (0, 2, 1))
    priors_t = priors.T

    out_l, out_c = pl.pallas_call(
        _mbox_kernel,
        grid=(B,),
        in_specs=[
            pl.BlockSpec((1, C, P), lambda b: (b, 0, 0)),
            pl.BlockSpec((1, 4, P), lambda b: (b, 0, 0)),
            pl.BlockSpec((4, P), lambda b: (0, 0)),
            pl.BlockSpec((1, 8, 5), lambda b: (b, 0, 0)),
        ],
        out_specs=[
            pl.BlockSpec((1, 1), lambda b: (0, 0)),
            pl.BlockSpec((1, 1), lambda b: (0, 0)),
        ],
        out_shape=[
            jax.ShapeDtypeStruct((1, 1), jnp.float32),
            jax.ShapeDtypeStruct((1, 1), jnp.float32),
        ],
        scratch_shapes=[
            pltpu.VMEM((B, P), jnp.float32),
            pltpu.VMEM((B, 1), jnp.int32),
        ],
        compiler_params=pltpu.CompilerParams(
            dimension_semantics=("arbitrary",),
        ),
    )(conf_t, loc_t, priors_t, targets)
    return (out_l[0, 0], out_c[0, 0])
